# baseline (speedup 1.0000x reference)
import functools

import jax
import jax.numpy as jnp
from jax.experimental import pallas as pl
from jax.experimental.pallas import tpu as pltpu

PRE_SEQ_LEN = 128
EMB_DIM = 18432
BATCH = 32
ROWS = BATCH * PRE_SEQ_LEN
SUB = EMB_DIM // 128
RB = 64


def _tc_gather(pref, table3):
    grid = ROWS // RB

    def body(pref_ref, t_ref, o_ref):
        i = pl.program_id(0)
        for r in range(RB):
            ix = pref_ref[i * RB + r]
            o_ref[r] = t_ref[ix]

    return pl.pallas_call(
        body,
        grid_spec=pltpu.PrefetchScalarGridSpec(
            num_scalar_prefetch=1,
            grid=(grid,),
            in_specs=[
                pl.BlockSpec((PRE_SEQ_LEN, SUB, 128), lambda i, p: (0, 0, 0)),
            ],
            out_specs=pl.BlockSpec((RB, SUB, 128), lambda i, p: (i, 0, 0)),
        ),
        out_shape=jax.ShapeDtypeStruct((ROWS, SUB, 128), jnp.float32),
    )(pref, table3)


def kernel(prefix, table):
    pref = prefix.astype(jnp.int32).reshape(ROWS)
    table3 = table.reshape(PRE_SEQ_LEN, SUB, 128)
    out = _tc_gather(pref, table3)
    return out.reshape(BATCH, PRE_SEQ_LEN, EMB_DIM)
